# baseline (speedup 1.0000x reference)
import functools

import jax
import jax.numpy as jnp
from jax import lax
from jax.experimental import pallas as pl
from jax.experimental.pallas import tpu as pltpu
from jax.experimental.pallas import tpu_sc as plsc

BATCH = 16384
FACTORS = 64

_info = plsc.get_sparse_core_info()
NC = _info.num_cores
NS = _info.num_subcores
NW = NC * NS
B_PER_W = BATCH // NW
IDX_CHUNK = 128
N_CHUNKS = B_PER_W // IDX_CHUNK


def _pointmf_kernel(user_hbm, item_hbm, eu_hbm, ei_hbm, out_hbm,
                    uidx_v, iidx_v, eu_v, ei_v, out_v, sem):
    wid = lax.axis_index("s") * NC + lax.axis_index("c")
    base = wid * B_PER_W

    pltpu.sync_copy(user_hbm.at[wid], uidx_v)
    pltpu.sync_copy(item_hbm.at[wid], iidx_v)

    copies = []
    for j in range(N_CHUNKS):
        rows = pl.ds(j * IDX_CHUNK, IDX_CHUNK)
        copies.append(pltpu.async_copy(eu_hbm.at[uidx_v.at[j]], eu_v.at[rows], sem))
        copies.append(pltpu.async_copy(ei_hbm.at[iidx_v.at[j]], ei_v.at[rows], sem))
    for c in copies:
        c.wait()

    lanes = lax.iota(jnp.int32, 16)

    def group(i, _):
        rows = i * 16 + lanes

        def fstep(f, acc):
            cols = (lanes + f) & 63
            a = plsc.load_gather(eu_v, [rows, cols])
            b = plsc.load_gather(ei_v, [rows, cols])
            return acc + a * b

        acc = lax.fori_loop(0, FACTORS, fstep, jnp.zeros((16,), jnp.float32),
                            unroll=8)
        out_v[pl.ds(i * 16, 16)] = acc
        return 0
    lax.fori_loop(0, B_PER_W // 16, group, 0)

    pltpu.sync_copy(out_v, out_hbm.at[pl.ds(base, B_PER_W)])


@jax.jit
def _run(user2d, item2d, embed_user, embed_item):
    mesh = plsc.VectorSubcoreMesh(core_axis_name="c", subcore_axis_name="s")
    f = functools.partial(
        pl.kernel,
        mesh=mesh,
        compiler_params=pltpu.CompilerParams(
            needs_layout_passes=False, use_tc_tiling_on_sc=False),
        out_type=jax.ShapeDtypeStruct((BATCH,), jnp.float32),
        scratch_types=[
            pltpu.VMEM((N_CHUNKS, IDX_CHUNK), jnp.int32),
            pltpu.VMEM((N_CHUNKS, IDX_CHUNK), jnp.int32),
            pltpu.VMEM((B_PER_W, FACTORS), jnp.float32),
            pltpu.VMEM((B_PER_W, FACTORS), jnp.float32),
            pltpu.VMEM((B_PER_W,), jnp.float32),
            pltpu.SemaphoreType.DMA,
        ],
    )(_pointmf_kernel)
    return f(user2d, item2d, embed_user, embed_item)


def kernel(user, item, embed_user, embed_item):
    user2d = user.astype(jnp.int32).reshape(NW, N_CHUNKS, IDX_CHUNK)
    item2d = item.astype(jnp.int32).reshape(NW, N_CHUNKS, IDX_CHUNK)
    return _run(user2d, item2d, embed_user, embed_item)
